# Initial kernel scaffold; baseline (speedup 1.0000x reference)
import jax
import jax.numpy as jnp
from jax.experimental import pallas as pl


def kernel(group_inputs, item_inputs, members, user_table, item_table, group_table, W1, b1, W2, b2, P1, pb1, P2, pb2):
    raise NotImplementedError("write your pallas kernel here")


# speedup vs baseline: 1.7337x; 1.7337x over previous
import functools

import jax
import jax.numpy as jnp
from jax import lax
from jax.experimental import pallas as pl
from jax.experimental.pallas import tpu as pltpu
from jax.experimental.pallas import tpu_sc as plsc

NC = 2
NS = 16
L = 16
NW = NC * NS

B = 1024
GS = 8
D = 128
BPW = B // NW
RPW = BPW * GS


def _sc_gather(group_inputs, item_inputs, members_flat, user_table,
               item_table, group_table):
    nmem = members_flat.shape[0]
    mesh = plsc.VectorSubcoreMesh(core_axis_name="c", subcore_axis_name="s")

    @functools.partial(
        pl.kernel,
        mesh=mesh,
        compiler_params=pltpu.CompilerParams(needs_layout_passes=False),
        out_type=[
            jax.ShapeDtypeStruct((B * GS, D), jnp.float32),
            jax.ShapeDtypeStruct((B, D), jnp.float32),
            jax.ShapeDtypeStruct((B, D), jnp.float32),
        ],
        scratch_types=[
            pltpu.VMEM((BPW,), jnp.int32),
            pltpu.VMEM((BPW,), jnp.int32),
            pltpu.VMEM((nmem,), jnp.int32),
            pltpu.VMEM((2, 128), jnp.int32),
            pltpu.VMEM((RPW, D), jnp.float32),
            pltpu.VMEM((BPW, D), jnp.float32),
            pltpu.VMEM((BPW, D), jnp.float32),
            pltpu.SemaphoreType.DMA,
        ],
    )
    def k(g_hbm, it_hbm, mem_hbm, ut_hbm, itab_hbm, gtab_hbm,
          urows_out, irows_out, grows_out,
          g_v, it_v, memtab_v, mem_idx_v, urows_v, irows_v, grows_v, sem):
        wid = lax.axis_index("s") * NC + lax.axis_index("c")
        base = wid * BPW
        pltpu.sync_copy(g_hbm.at[pl.ds(base, BPW)], g_v)
        pltpu.sync_copy(it_hbm.at[pl.ds(base, BPW)], it_v)
        pltpu.sync_copy(mem_hbm, memtab_v)
        lane = lax.iota(jnp.int32, L)
        for v in range(RPW // L):
            i = lane + v * L
            b_loc = lax.shift_right_logical(i, 3)
            j = lax.bitwise_and(i, GS - 1)
            g = plsc.load_gather(g_v, [b_loc])
            mem = plsc.load_gather(memtab_v, [g * GS + j])
            mem_idx_v[v // 8, pl.ds((v % 8) * L, L)] = mem
        c0 = pltpu.async_copy(ut_hbm.at[mem_idx_v.at[0]],
                              urows_v.at[pl.ds(0, 128)], sem)
        c1 = pltpu.async_copy(ut_hbm.at[mem_idx_v.at[1]],
                              urows_v.at[pl.ds(128, 128)], sem)
        c2 = pltpu.async_copy(itab_hbm.at[it_v], irows_v, sem)
        c3 = pltpu.async_copy(gtab_hbm.at[g_v], grows_v, sem)
        c0.wait()
        c1.wait()
        c2.wait()
        c3.wait()
        pltpu.sync_copy(urows_v, urows_out.at[pl.ds(base * GS, RPW)])
        pltpu.sync_copy(irows_v, irows_out.at[pl.ds(base, BPW)])
        pltpu.sync_copy(grows_v, grows_out.at[pl.ds(base, BPW)])

    return k(group_inputs, item_inputs, members_flat, user_table,
             item_table, group_table)


BBLK = 256


def _tc_body(u_ref, i_ref, g_ref, w1u_ref, w1i_ref, b1_ref, w2_ref, b2_ref,
             p1_ref, pb1_ref, p2_ref, pb2_ref, out_ref):
    u = u_ref[...]
    v = i_ref[...]
    hu = jnp.dot(u, w1u_ref[...], preferred_element_type=jnp.float32)
    hv = jnp.dot(v, w1i_ref[...], preferred_element_type=jnp.float32)
    h = hu.reshape(BBLK, GS, 16) + hv[:, None, :] + b1_ref[...][None]
    h = jnp.maximum(h, 0.0)
    scores = jnp.sum(h * w2_ref[...][None], axis=-1) + b2_ref[0, 0]
    m = jnp.max(scores, axis=-1, keepdims=True)
    e = jnp.exp(scores - m)
    at_wt = e / jnp.sum(e, axis=-1, keepdims=True)
    g_att = jnp.sum(at_wt[..., None] * u.reshape(BBLK, GS, D), axis=1)
    x = (g_att + g_ref[...]) * v
    t = jnp.dot(x, p1_ref[...], preferred_element_type=jnp.float32)
    t = jnp.maximum(t + pb1_ref[...], 0.0)
    p = jnp.sum(t * p2_ref[...], axis=-1, keepdims=True) + pb2_ref[0, 0]
    out_ref[...] = 1.0 / (1.0 + jnp.exp(-p))


def _tc_compute(urows, irows, grows, w1u, w1i, b1, w2, b2, p1, pb1, p2, pb2):
    nblk = B // BBLK
    full = lambda i: (0, 0)
    return pl.pallas_call(
        _tc_body,
        grid=(nblk,),
        in_specs=[
            pl.BlockSpec((BBLK * GS, D), lambda i: (i, 0)),
            pl.BlockSpec((BBLK, D), lambda i: (i, 0)),
            pl.BlockSpec((BBLK, D), lambda i: (i, 0)),
            pl.BlockSpec((D, 16), full),
            pl.BlockSpec((D, 16), full),
            pl.BlockSpec((1, 16), full),
            pl.BlockSpec((1, 16), full),
            pl.BlockSpec((1, 1), full),
            pl.BlockSpec((D, 8), full),
            pl.BlockSpec((1, 8), full),
            pl.BlockSpec((1, 8), full),
            pl.BlockSpec((1, 1), full),
        ],
        out_specs=pl.BlockSpec((BBLK, 1), lambda i: (i, 0)),
        out_shape=jax.ShapeDtypeStruct((B, 1), jnp.float32),
    )(urows, irows, grows, w1u, w1i, b1, w2, b2, p1, pb1, p2, pb2)


def kernel(group_inputs, item_inputs, members, user_table, item_table,
           group_table, W1, b1, W2, b2, P1, pb1, P2, pb2):
    urows, irows, grows = _sc_gather(
        group_inputs, item_inputs, members.reshape(-1),
        user_table, item_table, group_table)
    return _tc_compute(
        urows, irows, grows,
        W1[:D], W1[D:], b1.reshape(1, 16), W2.reshape(1, 16),
        b2.reshape(1, 1), P1, pb1.reshape(1, 8), P2.reshape(1, 8),
        pb2.reshape(1, 1))


# speedup vs baseline: 2.2002x; 1.2691x over previous
import functools

import jax
import jax.numpy as jnp
from jax import lax
from jax.experimental import pallas as pl
from jax.experimental.pallas import tpu as pltpu
from jax.experimental.pallas import tpu_sc as plsc

NC = 2
NS = 16
NW = NC * NS

B = 1024
NG = 64
GS = 8
D = 128
NM = NG * GS
MPW = NM // NW
BPW = B // NW


def _sc_gather(mem_t, item_inputs, user_table, item_table):
    mesh = plsc.VectorSubcoreMesh(core_axis_name="c", subcore_axis_name="s")

    @functools.partial(
        pl.kernel,
        mesh=mesh,
        compiler_params=pltpu.CompilerParams(needs_layout_passes=False),
        out_type=[
            jax.ShapeDtypeStruct((NM, D), jnp.float32),
            jax.ShapeDtypeStruct((B, D), jnp.float32),
        ],
        scratch_types=[
            pltpu.VMEM((MPW,), jnp.int32),
            pltpu.VMEM((BPW,), jnp.int32),
            pltpu.VMEM((MPW, D), jnp.float32),
            pltpu.VMEM((BPW, D), jnp.float32),
            pltpu.SemaphoreType.DMA,
            pltpu.SemaphoreType.DMA,
        ],
    )
    def k(memt_hbm, it_hbm, ut_hbm, itab_hbm, mrows_out, irows_out,
          uidx_v, iidx_v, mrows_v, irows_v, sem, sem2):
        wid = lax.axis_index("s") * NC + lax.axis_index("c")
        pltpu.sync_copy(memt_hbm.at[pl.ds(wid * MPW, MPW)], uidx_v)
        pltpu.sync_copy(it_hbm.at[pl.ds(wid * BPW, BPW)], iidx_v)
        c0 = pltpu.async_copy(ut_hbm.at[uidx_v], mrows_v, sem)
        c1 = pltpu.async_copy(itab_hbm.at[iidx_v], irows_v, sem)
        c0.wait()
        c1.wait()
        w0 = pltpu.async_copy(mrows_v, mrows_out.at[pl.ds(wid * MPW, MPW)],
                              sem2)
        w1 = pltpu.async_copy(irows_v, irows_out.at[pl.ds(wid * BPW, BPW)],
                              sem2)
        w0.wait()
        w1.wait()

    return k(mem_t, item_inputs, user_table, item_table)


BBLK = 256


def _tc_body(gf_ref, i_ref, m_ref, g64_ref, w1u_ref, w1i_ref, b1_ref, w2_ref,
             b2_ref, p1_ref, pb1_ref, p2_ref, pb2_ref, out_ref):
    gf = gf_ref[...]
    v = i_ref[...]
    mrows = m_ref[...]
    iota = lax.broadcasted_iota(jnp.int32, (BBLK, NG), 1).astype(jnp.float32)
    onehot = (gf == iota).astype(jnp.float32)
    hv = jnp.dot(v, w1i_ref[...], preferred_element_type=jnp.float32)
    hu_all = jnp.dot(mrows, w1u_ref[...],
                     preferred_element_type=jnp.float32)
    b1 = b1_ref[...]
    w2 = w2_ref[...]
    score_parts = []
    for j in range(GS):
        hu_j = jnp.dot(onehot, hu_all[j * NG:(j + 1) * NG, :],
                       preferred_element_type=jnp.float32)
        h_j = jnp.maximum(hu_j + hv + b1, 0.0)
        score_parts.append(jnp.sum(h_j * w2, axis=-1, keepdims=True))
    scores = jnp.concatenate(score_parts, axis=-1) + b2_ref[0, 0]
    m = jnp.max(scores, axis=-1, keepdims=True)
    e = jnp.exp(scores - m)
    at_wt = e / jnp.sum(e, axis=-1, keepdims=True)
    acat = jnp.concatenate(
        [onehot * at_wt[:, j:j + 1] for j in range(GS)], axis=-1)
    g_att = jnp.dot(acat, mrows, preferred_element_type=jnp.float32)
    grp = jnp.dot(onehot, g64_ref[...], preferred_element_type=jnp.float32)
    x = (g_att + grp) * v
    t = jnp.dot(x, p1_ref[...], preferred_element_type=jnp.float32)
    t = jnp.maximum(t + pb1_ref[...], 0.0)
    p = jnp.sum(t * p2_ref[...], axis=-1, keepdims=True) + pb2_ref[0, 0]
    out_ref[...] = 1.0 / (1.0 + jnp.exp(-p))


def _tc_compute(gf, irows, mrows, group_table, w1u, w1i, b1, w2, b2,
                p1, pb1, p2, pb2):
    nblk = B // BBLK
    full = lambda i: (0, 0)
    return pl.pallas_call(
        _tc_body,
        grid=(nblk,),
        in_specs=[
            pl.BlockSpec((BBLK, 1), lambda i: (i, 0)),
            pl.BlockSpec((BBLK, D), lambda i: (i, 0)),
            pl.BlockSpec((NM, D), full),
            pl.BlockSpec((NG, D), full),
            pl.BlockSpec((D, 16), full),
            pl.BlockSpec((D, 16), full),
            pl.BlockSpec((1, 16), full),
            pl.BlockSpec((1, 16), full),
            pl.BlockSpec((1, 1), full),
            pl.BlockSpec((D, 8), full),
            pl.BlockSpec((1, 8), full),
            pl.BlockSpec((1, 8), full),
            pl.BlockSpec((1, 1), full),
        ],
        out_specs=pl.BlockSpec((BBLK, 1), lambda i: (i, 0)),
        out_shape=jax.ShapeDtypeStruct((B, 1), jnp.float32),
    )(gf, irows, mrows, group_table, w1u, w1i, b1, w2, b2, p1, pb1, p2, pb2)


def kernel(group_inputs, item_inputs, members, user_table, item_table,
           group_table, W1, b1, W2, b2, P1, pb1, P2, pb2):
    mem_t = members.T.reshape(-1)
    mrows, irows = _sc_gather(mem_t, item_inputs, user_table, item_table)
    gf = group_inputs.astype(jnp.float32).reshape(B, 1)
    return _tc_compute(
        gf, irows, mrows, group_table,
        W1[:D], W1[D:], b1.reshape(1, 16), W2.reshape(1, 16),
        b2.reshape(1, 1), P1, pb1.reshape(1, 8), P2.reshape(1, 8),
        pb2.reshape(1, 1))


# speedup vs baseline: 2.2154x; 1.0069x over previous
import functools

import jax
import jax.numpy as jnp
from jax import lax
from jax.experimental import pallas as pl
from jax.experimental.pallas import tpu as pltpu
from jax.experimental.pallas import tpu_sc as plsc

NC = 2
NS = 16
NW = NC * NS

B = 1024
NG = 64
GS = 8
D = 128
NM = NG * GS
MPW = NM // NW
BPW = B // NW


def _sc_gather(mem_flat, item_inputs, user_table, item_table):
    mesh = plsc.VectorSubcoreMesh(core_axis_name="c", subcore_axis_name="s")

    @functools.partial(
        pl.kernel,
        mesh=mesh,
        compiler_params=pltpu.CompilerParams(needs_layout_passes=False),
        out_type=[
            jax.ShapeDtypeStruct((NM, D), jnp.float32),
            jax.ShapeDtypeStruct((B, D), jnp.float32),
        ],
        scratch_types=[
            pltpu.VMEM((MPW,), jnp.int32),
            pltpu.VMEM((BPW,), jnp.int32),
            pltpu.VMEM((MPW, D), jnp.float32),
            pltpu.VMEM((BPW, D), jnp.float32),
            pltpu.SemaphoreType.DMA,
            pltpu.SemaphoreType.DMA,
        ],
    )
    def k(memf_hbm, it_hbm, ut_hbm, itab_hbm, mrows_out, irows_out,
          uidx_v, iidx_v, mrows_v, irows_v, sem, sem2):
        wid = lax.axis_index("s") * NC + lax.axis_index("c")
        pltpu.sync_copy(memf_hbm.at[pl.ds(wid * MPW, MPW)], uidx_v)
        pltpu.sync_copy(it_hbm.at[pl.ds(wid * BPW, BPW)], iidx_v)
        c0 = pltpu.async_copy(ut_hbm.at[uidx_v], mrows_v, sem)
        c1 = pltpu.async_copy(itab_hbm.at[iidx_v], irows_v, sem)
        c0.wait()
        c1.wait()
        w0 = pltpu.async_copy(mrows_v, mrows_out.at[pl.ds(wid * MPW, MPW)],
                              sem2)
        w1 = pltpu.async_copy(irows_v, irows_out.at[pl.ds(wid * BPW, BPW)],
                              sem2)
        w0.wait()
        w1.wait()

    return k(mem_flat, item_inputs, user_table, item_table)


def _tc_body(gi_ref, i_ref, mcat_ref, g64_ref, w1u_ref, w1i_ref, b1_ref,
             w2_ref, b2_ref, p1_ref, pb1_ref, p2_ref, pb2_ref, out_ref):
    gi = gi_ref[...]
    v = i_ref[...]
    mcat = mcat_ref[...]
    iota = lax.broadcasted_iota(jnp.int32, (B, NG), 1)
    onehot = (gi == iota).astype(jnp.float32)
    ublk = jnp.dot(onehot, mcat,
                   preferred_element_type=jnp.float32)
    hv = jnp.dot(v, w1i_ref[...], preferred_element_type=jnp.float32)
    b1 = b1_ref[...]
    w1u = w1u_ref[...]
    h_parts = []
    for j in range(GS):
        u_j = ublk[:, j * D:(j + 1) * D]
        h_j = jnp.dot(u_j, w1u, preferred_element_type=jnp.float32)
        h_parts.append(jnp.maximum(h_j + hv + b1, 0.0))
    hcat = jnp.concatenate(h_parts, axis=-1)
    sel = (lax.broadcasted_iota(jnp.int32, (GS * 16, GS), 0) // 16
           == lax.broadcasted_iota(jnp.int32, (GS * 16, GS), 1))
    w2big = jnp.concatenate([w2_ref[...].reshape(16, 1)] * GS, axis=0)
    w2sel = jnp.where(sel, w2big, 0.0)
    scores = jnp.dot(hcat, w2sel,
                     preferred_element_type=jnp.float32) + b2_ref[0, 0]
    m = jnp.max(scores, axis=-1, keepdims=True)
    e = jnp.exp(scores - m)
    at_wt = e / jnp.sum(e, axis=-1, keepdims=True)
    g_att = at_wt[:, 0:1] * ublk[:, 0:D]
    for j in range(1, GS):
        g_att = g_att + at_wt[:, j:j + 1] * ublk[:, j * D:(j + 1) * D]
    grp = jnp.dot(onehot, g64_ref[...], preferred_element_type=jnp.float32)
    x = (g_att + grp) * v
    t = jnp.dot(x, p1_ref[...], preferred_element_type=jnp.float32)
    t = jnp.maximum(t + pb1_ref[...], 0.0)
    p = jnp.sum(t * p2_ref[...], axis=-1, keepdims=True) + pb2_ref[0, 0]
    out_ref[...] = 1.0 / (1.0 + jnp.exp(-p))


def _tc_compute(gi, irows, mcat, group_table, w1u, w1i, b1, w2, b2,
                p1, pb1, p2, pb2):
    full = lambda i: (0, 0)
    return pl.pallas_call(
        _tc_body,
        grid=(1,),
        in_specs=[
            pl.BlockSpec((B, 1), full),
            pl.BlockSpec((B, D), full),
            pl.BlockSpec((NG, GS * D), full),
            pl.BlockSpec((NG, D), full),
            pl.BlockSpec((D, 16), full),
            pl.BlockSpec((D, 16), lambda i: (1, 0)),
            pl.BlockSpec((1, 16), full),
            pl.BlockSpec((1, 16), full),
            pl.BlockSpec((1, 1), full),
            pl.BlockSpec((D, 8), full),
            pl.BlockSpec((1, 8), full),
            pl.BlockSpec((1, 8), full),
            pl.BlockSpec((1, 1), full),
        ],
        out_specs=pl.BlockSpec((B, 1), full),
        out_shape=jax.ShapeDtypeStruct((B, 1), jnp.float32),
    )(gi, irows, mcat, group_table, w1u, w1i, b1, w2, b2, p1, pb1, p2, pb2)


def kernel(group_inputs, item_inputs, members, user_table, item_table,
           group_table, W1, b1, W2, b2, P1, pb1, P2, pb2):
    mrows, irows = _sc_gather(members.reshape(-1), item_inputs,
                              user_table, item_table)
    return _tc_compute(
        group_inputs.reshape(B, 1), irows, mrows.reshape(NG, GS * D),
        group_table,
        W1, W1, b1.reshape(1, 16), W2.reshape(1, 16),
        b2.reshape(1, 1), P1, pb1.reshape(1, 8), P2.reshape(1, 8),
        pb2.reshape(1, 1))
